# Initial kernel scaffold; baseline (speedup 1.0000x reference)
import jax
import jax.numpy as jnp
from jax.experimental import pallas as pl


def kernel(x, edge_index, e_w, idx, W1, b1, W2, b2):
    raise NotImplementedError("write your pallas kernel here")


# speedup vs baseline: 23.1634x; 23.1634x over previous
import functools

import jax
import jax.numpy as jnp
from jax import lax
from jax.experimental import pallas as pl
from jax.experimental.pallas import tpu as pltpu
from jax.experimental.pallas import tpu_sc as plsc

_N = 10000
_D = 128
_H = 64
_CLS = 40
_K = 10

_NP = 10240
_EP = 360448
_RPT = _NP // 16
_EPT = _EP // 16
_BLK = 1024
_NBLK = _EPT // _BLK
_DROWS = _EP // 128
_DRPW = _DROWS // 32
_PB = 320
_RB = 256
_NRB = _NP // _RB


def _sc_mesh():
    return plsc.VectorSubcoreMesh(core_axis_name="c", subcore_axis_name="s")


def _deg_body(dst2_h, degp_h, degsh, ones_v, didx, zc):
    c = lax.axis_index("c")
    s = lax.axis_index("s")
    w = s * 2 + c
    wrow = pl.multiple_of(w * _DRPW, 8)
    srow = pl.multiple_of(s * _RPT, 8)
    orow = pl.multiple_of(c * _NP + s * _RPT, 8)

    @pl.loop(0, 128)
    def _(r):
        ones_v[r, pl.ds(0, 16)] = jnp.ones((16,), jnp.float32)

    @pl.loop(0, _RPT)
    def _(r):
        zc[r, pl.ds(0, 16)] = jnp.zeros((16,), jnp.float32)

    pltpu.sync_copy(zc, degsh.at[pl.ds(srow, _RPT)])
    pltpu.sync_copy(dst2_h.at[pl.ds(wrow, _DRPW)], didx)
    plsc.subcore_barrier()

    @pl.loop(0, _DRPW)
    def _(j):
        pltpu.sync_copy(ones_v, degsh.at[didx.at[j]], add=True)

    plsc.subcore_barrier()
    pltpu.sync_copy(degsh.at[pl.ds(srow, _RPT)], zc)
    pltpu.sync_copy(zc, degp_h.at[pl.ds(orow, _RPT)])


def _make_deg():
    return pl.kernel(
        _deg_body,
        out_type=jax.ShapeDtypeStruct((2 * _NP, 16), jnp.float32),
        mesh=_sc_mesh(),
        compiler_params=pltpu.CompilerParams(use_tc_tiling_on_sc=False),
        scratch_types=[
            pltpu.VMEM_SHARED((_NP, 16), jnp.float32),
            pltpu.VMEM((128, 16), jnp.float32),
            pltpu.VMEM((_DRPW, 128), jnp.int32),
            pltpu.VMEM((_RPT, 16), jnp.float32),
        ],
    )


def _prep_body(xb, w1, b1r, degb, u0, u1, hd0, hd1, d2, sd):
    i = pl.program_id(0)
    degv = degb[...]
    deg = (degv[0] + degv[1])[:, 0:1]
    h = jnp.maximum(jnp.dot(xb[...], w1[...],
                            preferred_element_type=jnp.float32) + b1r[...], 0.0)
    pos = deg > 0.0
    dis = jnp.where(pos, lax.rsqrt(jnp.maximum(deg, 1e-30)), 0.0)
    rid = i * _RB + lax.broadcasted_iota(jnp.int32, (_RB, 1), 0)
    rmask = jnp.where(rid < _N, 1.0, 0.0)
    uu = dis * h * rmask
    u0[...] = uu[:, :32]
    u1[...] = uu[:, 32:]
    hd0[...] = 0.1 * uu[:, :32]
    hd1[...] = 0.1 * uu[:, 32:]
    d2[...] = jnp.broadcast_to(0.9 * dis * dis, (_RB, 32))
    sd[...] = jnp.broadcast_to(
        jnp.where(pos, jnp.sqrt(jnp.maximum(deg, 0.0)), 0.0), (_RB, _H))


def _make_prep():
    f32 = jnp.float32
    o = jax.ShapeDtypeStruct
    return pl.pallas_call(
        _prep_body,
        grid=(_NRB,),
        in_specs=[
            pl.BlockSpec((_RB, _D), lambda i: (i, 0)),
            pl.BlockSpec((_D, _H), lambda i: (0, 0)),
            pl.BlockSpec((1, _H), lambda i: (0, 0)),
            pl.BlockSpec((2, _RB, 16), lambda i: (0, i, 0)),
        ],
        out_specs=[
            pl.BlockSpec((_RB, 32), lambda i: (i, 0)),
            pl.BlockSpec((_RB, 32), lambda i: (i, 0)),
            pl.BlockSpec((_RB, 32), lambda i: (i, 0)),
            pl.BlockSpec((_RB, 32), lambda i: (i, 0)),
            pl.BlockSpec((_RB, 32), lambda i: (i, 0)),
            pl.BlockSpec((_RB, _H), lambda i: (i, 0)),
        ],
        out_shape=[
            o((_NP, 32), f32), o((_NP, 32), f32),
            o((_NP, 32), f32), o((_NP, 32), f32),
            o((_NP, 32), f32), o((_NP, _H), f32),
        ],
    )


def _prop_body(src_h, dst2_h, u0_h, u1_h, hd0_h, hd1_h, d2_h,
               uf0_h, uf1_h,
               u_sh, agg_sh, sidx, didx, rows, aggc, d2c, hdc, zc, sem):
    c = lax.axis_index("c")
    s = lax.axis_index("s")
    row0 = pl.multiple_of(s * _RPT, 8)

    def _psl(p):
        return pl.ds(pl.multiple_of(row0 + p * _PB, 8), _PB)

    @pl.loop(0, _PB)
    def _(r):
        z16 = jnp.zeros((16,), jnp.float32)
        zc[r, pl.ds(0, 16)] = z16
        zc[r, pl.ds(16, 16)] = z16

    for p in range(_RPT // _PB):
        psl = _psl(p)

        @pl.when(c == 0)
        def _():
            pltpu.sync_copy(u0_h.at[psl], aggc)

        @pl.when(c == 1)
        def _():
            pltpu.sync_copy(u1_h.at[psl], aggc)

        pltpu.sync_copy(aggc, u_sh.at[psl])
        pltpu.sync_copy(zc, agg_sh.at[psl])

    plsc.subcore_barrier()

    ebase = s * _EPT

    @pl.loop(0, _K)
    def _(k):
        @pl.loop(0, _NBLK)
        def _(b):
            base = pl.multiple_of(ebase + b * _BLK, 128)
            pltpu.sync_copy(src_h.at[pl.ds(base, _BLK)], sidx)
            pltpu.sync_copy(dst2_h.at[pl.ds(pl.multiple_of(base // 128, 8), 8)], didx)
            pltpu.sync_copy(u_sh.at[sidx], rows)
            descs = [
                pltpu.async_copy(rows.at[pl.ds(j * 128, 128)],
                                 agg_sh.at[didx.at[j]], sem, add=True)
                for j in range(8)
            ]
            for dsc in descs:
                dsc.wait()

        plsc.subcore_barrier()
        for p in range(_RPT // _PB):
            psl = _psl(p)
            pltpu.sync_copy(agg_sh.at[psl], aggc)
            pltpu.sync_copy(d2_h.at[psl], d2c)

            @pl.when(c == 0)
            def _():
                pltpu.sync_copy(hd0_h.at[psl], hdc)

            @pl.when(c == 1)
            def _():
                pltpu.sync_copy(hd1_h.at[psl], hdc)

            @pl.loop(0, _PB)
            def _(r):
                for cc in (0, 16):
                    sl = pl.ds(cc, 16)
                    aggc[r, sl] = d2c[r, sl] * aggc[r, sl] + hdc[r, sl]

            pltpu.sync_copy(aggc, u_sh.at[psl])
            pltpu.sync_copy(zc, agg_sh.at[psl])

        plsc.subcore_barrier()

    for p in range(_RPT // _PB):
        psl = _psl(p)
        pltpu.sync_copy(u_sh.at[psl], aggc)

        @pl.when(c == 0)
        def _():
            pltpu.sync_copy(aggc, uf0_h.at[psl])

        @pl.when(c == 1)
        def _():
            pltpu.sync_copy(aggc, uf1_h.at[psl])


def _make_prop():
    f32 = jnp.float32
    o = jax.ShapeDtypeStruct
    return pl.kernel(
        _prop_body,
        out_type=[o((_NP, 32), f32), o((_NP, 32), f32)],
        mesh=_sc_mesh(),
        compiler_params=pltpu.CompilerParams(use_tc_tiling_on_sc=False),
        scratch_types=[
            pltpu.VMEM_SHARED((_NP, 32), f32),
            pltpu.VMEM_SHARED((_NP, 32), f32),
            pltpu.VMEM((_BLK,), jnp.int32),
            pltpu.VMEM((8, 128), jnp.int32),
            pltpu.VMEM((_BLK, 32), f32),
            pltpu.VMEM((_PB, 32), f32),
            pltpu.VMEM((_PB, 32), f32),
            pltpu.VMEM((_PB, 32), f32),
            pltpu.VMEM((_PB, 32), f32),
            pltpu.SemaphoreType.DMA,
        ],
    )


def _final_body(u0b, u1b, sdb, w2, b2r, lsm, xo, sm):
    z = jnp.concatenate([u0b[...], u1b[...]], axis=1) * sdb[...]
    logits = jnp.dot(z, w2[...], preferred_element_type=jnp.float32) + b2r[...]
    m = jnp.max(logits, axis=1, keepdims=True)
    ex = jnp.exp(logits - m)
    ssum = jnp.sum(ex, axis=1, keepdims=True)
    xo[...] = logits
    lsm[...] = logits - m - jnp.log(ssum)
    sm[...] = ex / ssum


def _make_final():
    f32 = jnp.float32
    o = jax.ShapeDtypeStruct
    return pl.pallas_call(
        _final_body,
        grid=(_NRB,),
        in_specs=[
            pl.BlockSpec((_RB, 32), lambda i: (i, 0)),
            pl.BlockSpec((_RB, 32), lambda i: (i, 0)),
            pl.BlockSpec((_RB, _H), lambda i: (i, 0)),
            pl.BlockSpec((_H, 128), lambda i: (0, 0)),
            pl.BlockSpec((1, 128), lambda i: (0, 0)),
        ],
        out_specs=[
            pl.BlockSpec((_RB, 128), lambda i: (i, 0)),
            pl.BlockSpec((_RB, 128), lambda i: (i, 0)),
            pl.BlockSpec((_RB, 128), lambda i: (i, 0)),
        ],
        out_shape=[o((_NP, 128), f32), o((_NP, 128), f32), o((_NP, 128), f32)],
    )


def kernel(x, edge_index, e_w, idx, W1, b1, W2, b2):
    del e_w, idx
    n_extra = _EP - (edge_index.shape[1] + _N)
    loops = jnp.arange(_N, dtype=jnp.int32)
    padv = _N + (jnp.arange(n_extra, dtype=jnp.int32) % (_NP - _N))
    src = jnp.concatenate([edge_index[0], loops, padv])
    dst = jnp.concatenate([edge_index[1], loops, padv])
    dst2 = dst.reshape(_DROWS, 128)

    xp = jnp.pad(x, ((0, _NP - _N), (0, 0)))
    b1r = b1.reshape(1, _H)
    w2p = jnp.pad(W2, ((0, 0), (0, 128 - _CLS)))
    b2r = jnp.concatenate(
        [b2, jnp.full((128 - _CLS,), -1e30, jnp.float32)]).reshape(1, 128)

    degp = _make_deg()(dst2).reshape(2, _NP, 16)
    u0, u1, hd0, hd1, d2, sd = _make_prep()(xp, W1, b1r, degp)
    uf0, uf1 = _make_prop()(src, dst2, u0, u1, hd0, hd1, d2)
    lsm, xo, sm = _make_final()(uf0, uf1, sd, w2p, b2r)
    return (lsm[:_N, :_CLS], xo[:_N, :_CLS], 0.0, sm[:_N, :_CLS])


# speedup vs baseline: 25.9220x; 1.1191x over previous
import functools

import jax
import jax.numpy as jnp
from jax import lax
from jax.experimental import pallas as pl
from jax.experimental.pallas import tpu as pltpu
from jax.experimental.pallas import tpu_sc as plsc

_N = 10000
_D = 128
_H = 64
_CLS = 40
_K = 10

_NP = 10240
_EP = 360448
_RPT = _NP // 16
_EPT = _EP // 16
_BLK = 1024
_NBLK = _EPT // _BLK
_DROWS = _EP // 128
_DRPW = _DROWS // 32
_PB = 160
_NPAIR = _NBLK // 2
_RB = 256
_NRB = _NP // _RB


def _sc_mesh():
    return plsc.VectorSubcoreMesh(core_axis_name="c", subcore_axis_name="s")


def _deg_body(dst2_h, degp_h, degsh, ones_v, didx, zc, semd):
    c = lax.axis_index("c")
    s = lax.axis_index("s")
    w = s * 2 + c
    wrow = pl.multiple_of(w * _DRPW, 8)
    srow = pl.multiple_of(s * _RPT, 8)
    orow = pl.multiple_of(c * _NP + s * _RPT, 8)

    @pl.loop(0, 128)
    def _(r):
        ones_v[r, pl.ds(0, 16)] = jnp.ones((16,), jnp.float32)

    @pl.loop(0, _RPT)
    def _(r):
        zc[r, pl.ds(0, 16)] = jnp.zeros((16,), jnp.float32)

    pltpu.sync_copy(zc, degsh.at[pl.ds(srow, _RPT)])
    pltpu.sync_copy(dst2_h.at[pl.ds(wrow, _DRPW)], didx)
    plsc.subcore_barrier()

    @pl.loop(0, _DRPW // 8)
    def _(jj):
        descs = [
            pltpu.async_copy(ones_v, degsh.at[didx.at[jj * 8 + u]],
                             semd, add=True)
            for u in range(8)
        ]
        for dsc in descs:
            dsc.wait()

    plsc.subcore_barrier()
    pltpu.sync_copy(degsh.at[pl.ds(srow, _RPT)], zc)
    pltpu.sync_copy(zc, degp_h.at[pl.ds(orow, _RPT)])


def _make_deg():
    return pl.kernel(
        _deg_body,
        out_type=jax.ShapeDtypeStruct((2 * _NP, 16), jnp.float32),
        mesh=_sc_mesh(),
        compiler_params=pltpu.CompilerParams(use_tc_tiling_on_sc=False),
        scratch_types=[
            pltpu.VMEM_SHARED((_NP, 16), jnp.float32),
            pltpu.VMEM((128, 16), jnp.float32),
            pltpu.VMEM((_DRPW, 128), jnp.int32),
            pltpu.VMEM((_RPT, 16), jnp.float32),
            pltpu.SemaphoreType.DMA,
        ],
    )


def _prep_body(xb, w1, b1r, degb, u0, u1, hd0, hd1, d2, sd):
    i = pl.program_id(0)
    degv = degb[...]
    deg = (degv[0] + degv[1])[:, 0:1]
    h = jnp.maximum(jnp.dot(xb[...], w1[...],
                            preferred_element_type=jnp.float32) + b1r[...], 0.0)
    pos = deg > 0.0
    dis = jnp.where(pos, lax.rsqrt(jnp.maximum(deg, 1e-30)), 0.0)
    rid = i * _RB + lax.broadcasted_iota(jnp.int32, (_RB, 1), 0)
    rmask = jnp.where(rid < _N, 1.0, 0.0)
    uu = dis * h * rmask
    u0[...] = uu[:, :32]
    u1[...] = uu[:, 32:]
    hd0[...] = 0.1 * uu[:, :32]
    hd1[...] = 0.1 * uu[:, 32:]
    d2[...] = jnp.broadcast_to(0.9 * dis * dis, (_RB, 32))
    sd[...] = jnp.broadcast_to(
        jnp.where(pos, jnp.sqrt(jnp.maximum(deg, 0.0)), 0.0), (_RB, _H))


def _make_prep():
    f32 = jnp.float32
    o = jax.ShapeDtypeStruct
    return pl.pallas_call(
        _prep_body,
        grid=(_NRB,),
        in_specs=[
            pl.BlockSpec((_RB, _D), lambda i: (i, 0)),
            pl.BlockSpec((_D, _H), lambda i: (0, 0)),
            pl.BlockSpec((1, _H), lambda i: (0, 0)),
            pl.BlockSpec((2, _RB, 16), lambda i: (0, i, 0)),
        ],
        out_specs=[
            pl.BlockSpec((_RB, 32), lambda i: (i, 0)),
            pl.BlockSpec((_RB, 32), lambda i: (i, 0)),
            pl.BlockSpec((_RB, 32), lambda i: (i, 0)),
            pl.BlockSpec((_RB, 32), lambda i: (i, 0)),
            pl.BlockSpec((_RB, 32), lambda i: (i, 0)),
            pl.BlockSpec((_RB, _H), lambda i: (i, 0)),
        ],
        out_shape=[
            o((_NP, 32), f32), o((_NP, 32), f32),
            o((_NP, 32), f32), o((_NP, 32), f32),
            o((_NP, 32), f32), o((_NP, _H), f32),
        ],
    )


def _prop_body(src_h, dst2_h, u0_h, u1_h, hd0_h, hd1_h, d2_h,
               uf0_h, uf1_h,
               u_sh, agg_sh, sidx0, sidx1, didx, rows0, rows1,
               aggc, d2c, hdc, zc, semg0, semg1, sems):
    c = lax.axis_index("c")
    s = lax.axis_index("s")
    row0 = pl.multiple_of(s * _RPT, 8)

    def _psl(p):
        return pl.ds(pl.multiple_of(row0 + p * _PB, 8), _PB)

    @pl.loop(0, _PB)
    def _(r):
        z16 = jnp.zeros((16,), jnp.float32)
        zc[r, pl.ds(0, 16)] = z16
        zc[r, pl.ds(16, 16)] = z16

    for p in range(_RPT // _PB):
        psl = _psl(p)

        @pl.when(c == 0)
        def _():
            pltpu.sync_copy(u0_h.at[psl], aggc)

        @pl.when(c == 1)
        def _():
            pltpu.sync_copy(u1_h.at[psl], aggc)

        pltpu.sync_copy(aggc, u_sh.at[psl])
        pltpu.sync_copy(zc, agg_sh.at[psl])

    plsc.subcore_barrier()

    ebase = s * _EPT

    def _scatter8(rows, sidx, semg):
        pltpu.make_async_copy(u_sh.at[sidx], rows, semg).wait()
        descs = [
            pltpu.async_copy(rows.at[pl.ds(j * 128, 128)],
                             agg_sh.at[didx.at[j]], sems, add=True)
            for j in range(8)
        ]
        for dsc in descs:
            dsc.wait()

    @pl.loop(0, _K)
    def _(k):
        b0p = pl.multiple_of(ebase, 128)
        pltpu.sync_copy(src_h.at[pl.ds(b0p, _BLK)], sidx0)
        pltpu.async_copy(u_sh.at[sidx0], rows0, semg0)

        @pl.loop(0, _NPAIR)
        def _(i):
            b0 = pl.multiple_of(ebase + i * 2 * _BLK, 128)
            b1 = pl.multiple_of(b0 + _BLK, 128)
            pltpu.sync_copy(src_h.at[pl.ds(b1, _BLK)], sidx1)
            pltpu.async_copy(u_sh.at[sidx1], rows1, semg1)
            pltpu.sync_copy(
                dst2_h.at[pl.ds(pl.multiple_of(b0 // 128, 8), 8)], didx)
            _scatter8(rows0, sidx0, semg0)

            @pl.when(i + 1 < _NPAIR)
            def _():
                b2 = pl.multiple_of(b0 + 2 * _BLK, 128)
                pltpu.sync_copy(src_h.at[pl.ds(b2, _BLK)], sidx0)
                pltpu.async_copy(u_sh.at[sidx0], rows0, semg0)

            pltpu.sync_copy(
                dst2_h.at[pl.ds(pl.multiple_of(b1 // 128, 8), 8)], didx)
            _scatter8(rows1, sidx1, semg1)

        plsc.subcore_barrier()
        for p in range(_RPT // _PB):
            psl = _psl(p)
            pltpu.sync_copy(agg_sh.at[psl], aggc)
            pltpu.sync_copy(d2_h.at[psl], d2c)

            @pl.when(c == 0)
            def _():
                pltpu.sync_copy(hd0_h.at[psl], hdc)

            @pl.when(c == 1)
            def _():
                pltpu.sync_copy(hd1_h.at[psl], hdc)

            @pl.loop(0, _PB)
            def _(r):
                for cc in (0, 16):
                    sl = pl.ds(cc, 16)
                    aggc[r, sl] = d2c[r, sl] * aggc[r, sl] + hdc[r, sl]

            pltpu.sync_copy(aggc, u_sh.at[psl])
            pltpu.sync_copy(zc, agg_sh.at[psl])

        plsc.subcore_barrier()

    for p in range(_RPT // _PB):
        psl = _psl(p)
        pltpu.sync_copy(u_sh.at[psl], aggc)

        @pl.when(c == 0)
        def _():
            pltpu.sync_copy(aggc, uf0_h.at[psl])

        @pl.when(c == 1)
        def _():
            pltpu.sync_copy(aggc, uf1_h.at[psl])


def _make_prop():
    f32 = jnp.float32
    o = jax.ShapeDtypeStruct
    return pl.kernel(
        _prop_body,
        out_type=[o((_NP, 32), f32), o((_NP, 32), f32)],
        mesh=_sc_mesh(),
        compiler_params=pltpu.CompilerParams(use_tc_tiling_on_sc=False),
        scratch_types=[
            pltpu.VMEM_SHARED((_NP, 32), f32),
            pltpu.VMEM_SHARED((_NP, 32), f32),
            pltpu.VMEM((_BLK,), jnp.int32),
            pltpu.VMEM((_BLK,), jnp.int32),
            pltpu.VMEM((8, 128), jnp.int32),
            pltpu.VMEM((_BLK, 32), f32),
            pltpu.VMEM((_BLK, 32), f32),
            pltpu.VMEM((_PB, 32), f32),
            pltpu.VMEM((_PB, 32), f32),
            pltpu.VMEM((_PB, 32), f32),
            pltpu.VMEM((_PB, 32), f32),
            pltpu.SemaphoreType.DMA,
            pltpu.SemaphoreType.DMA,
            pltpu.SemaphoreType.DMA,
        ],
    )


def _final_body(u0b, u1b, sdb, w2, b2r, lsm, xo, sm):
    z = jnp.concatenate([u0b[...], u1b[...]], axis=1) * sdb[...]
    logits = jnp.dot(z, w2[...], preferred_element_type=jnp.float32) + b2r[...]
    m = jnp.max(logits, axis=1, keepdims=True)
    ex = jnp.exp(logits - m)
    ssum = jnp.sum(ex, axis=1, keepdims=True)
    xo[...] = logits
    lsm[...] = logits - m - jnp.log(ssum)
    sm[...] = ex / ssum


def _make_final():
    f32 = jnp.float32
    o = jax.ShapeDtypeStruct
    return pl.pallas_call(
        _final_body,
        grid=(_NRB,),
        in_specs=[
            pl.BlockSpec((_RB, 32), lambda i: (i, 0)),
            pl.BlockSpec((_RB, 32), lambda i: (i, 0)),
            pl.BlockSpec((_RB, _H), lambda i: (i, 0)),
            pl.BlockSpec((_H, 128), lambda i: (0, 0)),
            pl.BlockSpec((1, 128), lambda i: (0, 0)),
        ],
        out_specs=[
            pl.BlockSpec((_RB, 128), lambda i: (i, 0)),
            pl.BlockSpec((_RB, 128), lambda i: (i, 0)),
            pl.BlockSpec((_RB, 128), lambda i: (i, 0)),
        ],
        out_shape=[o((_NP, 128), f32), o((_NP, 128), f32), o((_NP, 128), f32)],
    )


def kernel(x, edge_index, e_w, idx, W1, b1, W2, b2):
    del e_w, idx
    n_extra = _EP - (edge_index.shape[1] + _N)
    loops = jnp.arange(_N, dtype=jnp.int32)
    padv = _N + (jnp.arange(n_extra, dtype=jnp.int32) % (_NP - _N))
    src = jnp.concatenate([edge_index[0], loops, padv])
    dst = jnp.concatenate([edge_index[1], loops, padv])
    dst2 = dst.reshape(_DROWS, 128)

    xp = jnp.pad(x, ((0, _NP - _N), (0, 0)))
    b1r = b1.reshape(1, _H)
    w2p = jnp.pad(W2, ((0, 0), (0, 128 - _CLS)))
    b2r = jnp.concatenate(
        [b2, jnp.full((128 - _CLS,), -1e30, jnp.float32)]).reshape(1, 128)

    degp = _make_deg()(dst2).reshape(2, _NP, 16)
    u0, u1, hd0, hd1, d2, sd = _make_prep()(xp, W1, b1r, degp)
    uf0, uf1 = _make_prop()(src, dst2, u0, u1, hd0, hd1, d2)
    lsm, xo, sm = _make_final()(uf0, uf1, sd, w2p, b2r)
    return (lsm[:_N, :_CLS], xo[:_N, :_CLS], 0.0, sm[:_N, :_CLS])


# speedup vs baseline: 30.7937x; 1.1879x over previous
import functools

import jax
import jax.numpy as jnp
from jax import lax
from jax.experimental import pallas as pl
from jax.experimental.pallas import tpu as pltpu
from jax.experimental.pallas import tpu_sc as plsc

_N = 10000
_D = 128
_H = 64
_CLS = 40
_K = 10

_NP = 10240
_EP = 360448
_RPT = _NP // 16
_EPT = _EP // 16
_BLK = 1024
_NBLK = _EPT // _BLK
_DROWS = _EP // 128
_DRPW = _DROWS // 32
_PB = 128
_NPAIR = _NBLK // 2
_RB = 256
_NRB = _NP // _RB


def _sc_mesh():
    return plsc.VectorSubcoreMesh(core_axis_name="c", subcore_axis_name="s")


def _deg_body(dst2_h, degp_h, degsh, ones_v, didx, zc, semd):
    c = lax.axis_index("c")
    s = lax.axis_index("s")
    w = s * 2 + c
    wrow = pl.multiple_of(w * _DRPW, 8)
    srow = pl.multiple_of(s * _RPT, 8)
    orow = pl.multiple_of(c * _NP + s * _RPT, 8)

    @pl.loop(0, 128)
    def _(r):
        ones_v[r, pl.ds(0, 16)] = jnp.ones((16,), jnp.float32)

    @pl.loop(0, _RPT)
    def _(r):
        zc[r, pl.ds(0, 16)] = jnp.zeros((16,), jnp.float32)

    pltpu.sync_copy(zc, degsh.at[pl.ds(srow, _RPT)])
    pltpu.sync_copy(dst2_h.at[pl.ds(wrow, _DRPW)], didx)
    plsc.subcore_barrier()

    @pl.loop(0, _DRPW // 8)
    def _(jj):
        descs = [
            pltpu.async_copy(ones_v, degsh.at[didx.at[jj * 8 + u]],
                             semd, add=True)
            for u in range(8)
        ]
        for dsc in descs:
            dsc.wait()

    plsc.subcore_barrier()
    pltpu.sync_copy(degsh.at[pl.ds(srow, _RPT)], zc)
    pltpu.sync_copy(zc, degp_h.at[pl.ds(orow, _RPT)])


def _make_deg():
    return pl.kernel(
        _deg_body,
        out_type=jax.ShapeDtypeStruct((2 * _NP, 16), jnp.float32),
        mesh=_sc_mesh(),
        compiler_params=pltpu.CompilerParams(use_tc_tiling_on_sc=False),
        scratch_types=[
            pltpu.VMEM_SHARED((_NP, 16), jnp.float32),
            pltpu.VMEM((128, 16), jnp.float32),
            pltpu.VMEM((_DRPW, 128), jnp.int32),
            pltpu.VMEM((_RPT, 16), jnp.float32),
            pltpu.SemaphoreType.DMA,
        ],
    )


def _prep_body(xb, w1, b1r, degb, u0, u1, hd0, hd1, d2, sd):
    i = pl.program_id(0)
    degv = degb[...]
    deg = (degv[0] + degv[1])[:, 0:1]
    h = jnp.maximum(jnp.dot(xb[...], w1[...],
                            preferred_element_type=jnp.float32) + b1r[...], 0.0)
    pos = deg > 0.0
    dis = jnp.where(pos, lax.rsqrt(jnp.maximum(deg, 1e-30)), 0.0)
    rid = i * _RB + lax.broadcasted_iota(jnp.int32, (_RB, 1), 0)
    rmask = jnp.where(rid < _N, 1.0, 0.0)
    uu = dis * h * rmask
    u0[...] = uu[:, :32]
    u1[...] = uu[:, 32:]
    hd0[...] = 0.1 * uu[:, :32]
    hd1[...] = 0.1 * uu[:, 32:]
    d2[...] = jnp.broadcast_to(0.9 * dis * dis, (_RB, 32))
    sd[...] = jnp.broadcast_to(
        jnp.where(pos, jnp.sqrt(jnp.maximum(deg, 0.0)), 0.0), (_RB, _H))


def _make_prep():
    f32 = jnp.float32
    o = jax.ShapeDtypeStruct
    return pl.pallas_call(
        _prep_body,
        grid=(_NRB,),
        in_specs=[
            pl.BlockSpec((_RB, _D), lambda i: (i, 0)),
            pl.BlockSpec((_D, _H), lambda i: (0, 0)),
            pl.BlockSpec((1, _H), lambda i: (0, 0)),
            pl.BlockSpec((2, _RB, 16), lambda i: (0, i, 0)),
        ],
        out_specs=[
            pl.BlockSpec((_RB, 32), lambda i: (i, 0)),
            pl.BlockSpec((_RB, 32), lambda i: (i, 0)),
            pl.BlockSpec((_RB, 32), lambda i: (i, 0)),
            pl.BlockSpec((_RB, 32), lambda i: (i, 0)),
            pl.BlockSpec((_RB, 32), lambda i: (i, 0)),
            pl.BlockSpec((_RB, _H), lambda i: (i, 0)),
        ],
        out_shape=[
            o((_NP, 32), f32), o((_NP, 32), f32),
            o((_NP, 32), f32), o((_NP, 32), f32),
            o((_NP, 32), f32), o((_NP, _H), f32),
        ],
    )


def _prop_body(src_h, dst2_h, u0_h, u1_h, hd0_h, hd1_h, d2_h,
               uf0_h, uf1_h,
               agg_sh, sidx0, sidx1, didx0, didx1, rows0, rows1,
               aggc, d2c, hdc, zc, semg0, semg1, semi0, semi1, sems):
    c = lax.axis_index("c")
    s = lax.axis_index("s")
    row0 = pl.multiple_of(s * _RPT, 8)
    ebase = s * _EPT
    bufs = ((sidx0, didx0, rows0, semg0, semi0),
            (sidx1, didx1, rows1, semg1, semi1))

    def _psl(p):
        return pl.ds(pl.multiple_of(row0 + p * _PB, 8), _PB)

    def _eoff(b):
        return pl.multiple_of(ebase + b * _BLK, 128)

    def _idx_issue(boff, par):
        sidx, didx, _, _, semi = bufs[par]
        pltpu.async_copy(src_h.at[pl.ds(boff, _BLK)], sidx, semi)
        pltpu.async_copy(
            dst2_h.at[pl.ds(pl.multiple_of(boff // 128, 8), 8)], didx, semi)

    def _idx_wait(par):
        sidx, didx, _, _, semi = bufs[par]
        pltpu.make_async_copy(src_h.at[pl.ds(0, _BLK)], sidx, semi).wait()
        pltpu.make_async_copy(dst2_h.at[pl.ds(0, 8)], didx, semi).wait()

    def _gather_issue(par):
        sidx, _, rows, semg, _ = bufs[par]

        @pl.when(c == 0)
        def _():
            pltpu.async_copy(uf0_h.at[sidx], rows, semg)

        @pl.when(c == 1)
        def _():
            pltpu.async_copy(uf1_h.at[sidx], rows, semg)

    def _gather_wait(par):
        sidx, _, rows, semg, _ = bufs[par]
        pltpu.make_async_copy(uf0_h.at[sidx], rows, semg).wait()

    def _scatter8(par):
        _, didx, rows, _, _ = bufs[par]
        descs = [
            pltpu.async_copy(rows.at[pl.ds(j * 128, 128)],
                             agg_sh.at[didx.at[j]], sems, add=True)
            for j in range(8)
        ]
        for dsc in descs:
            dsc.wait()

    @pl.loop(0, _PB)
    def _(r):
        z16 = jnp.zeros((16,), jnp.float32)
        zc[r, pl.ds(0, 16)] = z16
        zc[r, pl.ds(16, 16)] = z16

    for p in range(_RPT // _PB):
        psl = _psl(p)

        @pl.when(c == 0)
        def _():
            pltpu.sync_copy(u0_h.at[psl], aggc)
            pltpu.sync_copy(aggc, uf0_h.at[psl])

        @pl.when(c == 1)
        def _():
            pltpu.sync_copy(u1_h.at[psl], aggc)
            pltpu.sync_copy(aggc, uf1_h.at[psl])

        pltpu.sync_copy(zc, agg_sh.at[psl])

    plsc.subcore_barrier()

    @pl.loop(0, _K)
    def _(k):
        pltpu.sync_copy(src_h.at[pl.ds(_eoff(0), _BLK)], sidx0)
        pltpu.sync_copy(
            dst2_h.at[pl.ds(pl.multiple_of(_eoff(0) // 128, 8), 8)], didx0)
        _gather_issue(0)
        _idx_issue(_eoff(1), 1)

        @pl.loop(0, _NPAIR)
        def _(i):
            b0 = 2 * i
            _idx_wait(1)
            _gather_issue(1)
            _gather_wait(0)
            _scatter8(0)

            @pl.when(i + 1 < _NPAIR)
            def _():
                _idx_issue(_eoff(b0 + 2), 0)

            @pl.when(i + 1 < _NPAIR)
            def _():
                _idx_wait(0)
                _gather_issue(0)

            _gather_wait(1)
            _scatter8(1)

            @pl.when(i + 1 < _NPAIR)
            def _():
                _idx_issue(_eoff(b0 + 3), 1)

        plsc.subcore_barrier()
        for p in range(_RPT // _PB):
            psl = _psl(p)
            pltpu.sync_copy(agg_sh.at[psl], aggc)
            pltpu.sync_copy(d2_h.at[psl], d2c)

            @pl.when(c == 0)
            def _():
                pltpu.sync_copy(hd0_h.at[psl], hdc)

            @pl.when(c == 1)
            def _():
                pltpu.sync_copy(hd1_h.at[psl], hdc)

            @pl.loop(0, _PB)
            def _(r):
                for cc in (0, 16):
                    sl = pl.ds(cc, 16)
                    aggc[r, sl] = d2c[r, sl] * aggc[r, sl] + hdc[r, sl]

            @pl.when(c == 0)
            def _():
                pltpu.sync_copy(aggc, uf0_h.at[psl])

            @pl.when(c == 1)
            def _():
                pltpu.sync_copy(aggc, uf1_h.at[psl])

            pltpu.sync_copy(zc, agg_sh.at[psl])

        plsc.subcore_barrier()


def _make_prop():
    f32 = jnp.float32
    o = jax.ShapeDtypeStruct
    return pl.kernel(
        _prop_body,
        out_type=[o((_NP, 32), f32), o((_NP, 32), f32)],
        mesh=_sc_mesh(),
        compiler_params=pltpu.CompilerParams(use_tc_tiling_on_sc=False),
        scratch_types=[
            pltpu.VMEM_SHARED((_NP, 32), f32),
            pltpu.VMEM((_BLK,), jnp.int32),
            pltpu.VMEM((_BLK,), jnp.int32),
            pltpu.VMEM((8, 128), jnp.int32),
            pltpu.VMEM((8, 128), jnp.int32),
            pltpu.VMEM((_BLK, 32), f32),
            pltpu.VMEM((_BLK, 32), f32),
            pltpu.VMEM((_PB, 32), f32),
            pltpu.VMEM((_PB, 32), f32),
            pltpu.VMEM((_PB, 32), f32),
            pltpu.VMEM((_PB, 32), f32),
            pltpu.SemaphoreType.DMA,
            pltpu.SemaphoreType.DMA,
            pltpu.SemaphoreType.DMA,
            pltpu.SemaphoreType.DMA,
            pltpu.SemaphoreType.DMA,
        ],
    )


def _final_body(u0b, u1b, sdb, w2, b2r, lsm, xo, sm):
    z = jnp.concatenate([u0b[...], u1b[...]], axis=1) * sdb[...]
    logits = jnp.dot(z, w2[...], preferred_element_type=jnp.float32) + b2r[...]
    m = jnp.max(logits, axis=1, keepdims=True)
    ex = jnp.exp(logits - m)
    ssum = jnp.sum(ex, axis=1, keepdims=True)
    xo[...] = logits
    lsm[...] = logits - m - jnp.log(ssum)
    sm[...] = ex / ssum


def _make_final():
    f32 = jnp.float32
    o = jax.ShapeDtypeStruct
    return pl.pallas_call(
        _final_body,
        grid=(_NRB,),
        in_specs=[
            pl.BlockSpec((_RB, 32), lambda i: (i, 0)),
            pl.BlockSpec((_RB, 32), lambda i: (i, 0)),
            pl.BlockSpec((_RB, _H), lambda i: (i, 0)),
            pl.BlockSpec((_H, 128), lambda i: (0, 0)),
            pl.BlockSpec((1, 128), lambda i: (0, 0)),
        ],
        out_specs=[
            pl.BlockSpec((_RB, 128), lambda i: (i, 0)),
            pl.BlockSpec((_RB, 128), lambda i: (i, 0)),
            pl.BlockSpec((_RB, 128), lambda i: (i, 0)),
        ],
        out_shape=[o((_NP, 128), f32), o((_NP, 128), f32), o((_NP, 128), f32)],
    )


def kernel(x, edge_index, e_w, idx, W1, b1, W2, b2):
    del e_w, idx
    n_extra = _EP - (edge_index.shape[1] + _N)
    loops = jnp.arange(_N, dtype=jnp.int32)
    padv = _N + (jnp.arange(n_extra, dtype=jnp.int32) % (_NP - _N))
    src = jnp.concatenate([edge_index[0], loops, padv])
    dst = jnp.concatenate([edge_index[1], loops, padv])
    dst2 = dst.reshape(_DROWS, 128)

    xp = jnp.pad(x, ((0, _NP - _N), (0, 0)))
    b1r = b1.reshape(1, _H)
    w2p = jnp.pad(W2, ((0, 0), (0, 128 - _CLS)))
    b2r = jnp.concatenate(
        [b2, jnp.full((128 - _CLS,), -1e30, jnp.float32)]).reshape(1, 128)

    degp = _make_deg()(dst2).reshape(2, _NP, 16)
    u0, u1, hd0, hd1, d2, sd = _make_prep()(xp, W1, b1r, degp)
    uf0, uf1 = _make_prop()(src, dst2, u0, u1, hd0, hd1, d2)
    lsm, xo, sm = _make_final()(uf0, uf1, sd, w2p, b2r)
    return (lsm[:_N, :_CLS], xo[:_N, :_CLS], 0.0, sm[:_N, :_CLS])


# speedup vs baseline: 32.3160x; 1.0494x over previous
import functools

import jax
import jax.numpy as jnp
from jax import lax
from jax.experimental import pallas as pl
from jax.experimental.pallas import tpu as pltpu
from jax.experimental.pallas import tpu_sc as plsc

_N = 10000
_D = 128
_H = 64
_CLS = 40
_K = 10

_NP = 10240
_EP = 360448
_RPT = _NP // 16
_EPT = _EP // 16
_BLK = 1024
_NBLK = _EPT // _BLK
_DROWS = _EP // 128
_DRPW = _DROWS // 32
_PB = 160
_NPAIR = _NBLK // 2
_RB = 256
_NRB = _NP // _RB


def _sc_mesh():
    return plsc.VectorSubcoreMesh(core_axis_name="c", subcore_axis_name="s")


def _deg_body(dst2_h, degp_h, degsh, ones_v, didx, zc, semd):
    c = lax.axis_index("c")
    s = lax.axis_index("s")
    w = s * 2 + c
    wrow = pl.multiple_of(w * _DRPW, 8)
    srow = pl.multiple_of(s * _RPT, 8)
    orow = pl.multiple_of(c * _NP + s * _RPT, 8)

    @pl.loop(0, 128)
    def _(r):
        ones_v[r, pl.ds(0, 16)] = jnp.ones((16,), jnp.float32)

    @pl.loop(0, _RPT)
    def _(r):
        zc[r, pl.ds(0, 16)] = jnp.zeros((16,), jnp.float32)

    pltpu.sync_copy(zc, degsh.at[pl.ds(srow, _RPT)])
    pltpu.sync_copy(dst2_h.at[pl.ds(wrow, _DRPW)], didx)
    plsc.subcore_barrier()

    @pl.loop(0, _DRPW // 8)
    def _(jj):
        descs = [
            pltpu.async_copy(ones_v, degsh.at[didx.at[jj * 8 + u]],
                             semd, add=True)
            for u in range(8)
        ]
        for dsc in descs:
            dsc.wait()

    plsc.subcore_barrier()
    pltpu.sync_copy(degsh.at[pl.ds(srow, _RPT)], zc)
    pltpu.sync_copy(zc, degp_h.at[pl.ds(orow, _RPT)])


def _make_deg():
    return pl.kernel(
        _deg_body,
        out_type=jax.ShapeDtypeStruct((2 * _NP, 16), jnp.float32),
        mesh=_sc_mesh(),
        compiler_params=pltpu.CompilerParams(use_tc_tiling_on_sc=False),
        scratch_types=[
            pltpu.VMEM_SHARED((_NP, 16), jnp.float32),
            pltpu.VMEM((128, 16), jnp.float32),
            pltpu.VMEM((_DRPW, 128), jnp.int32),
            pltpu.VMEM((_RPT, 16), jnp.float32),
            pltpu.SemaphoreType.DMA,
        ],
    )


def _prep_body(xb, w1, b1r, degb, u0, u1, hd0, hd1, d2, sd):
    i = pl.program_id(0)
    degv = degb[...]
    deg = (degv[0] + degv[1])[:, 0:1]
    h = jnp.maximum(jnp.dot(xb[...], w1[...],
                            preferred_element_type=jnp.float32) + b1r[...], 0.0)
    pos = deg > 0.0
    dis = jnp.where(pos, lax.rsqrt(jnp.maximum(deg, 1e-30)), 0.0)
    rid = i * _RB + lax.broadcasted_iota(jnp.int32, (_RB, 1), 0)
    rmask = jnp.where(rid < _N, 1.0, 0.0)
    uu = dis * h * rmask
    u0[...] = uu[:, :32]
    u1[...] = uu[:, 32:]
    hd0[...] = 0.1 * uu[:, :32]
    hd1[...] = 0.1 * uu[:, 32:]
    d2[...] = jnp.broadcast_to(0.9 * dis * dis, (_RB, 32))
    sd[...] = jnp.broadcast_to(
        jnp.where(pos, jnp.sqrt(jnp.maximum(deg, 0.0)), 0.0), (_RB, _H))


def _make_prep():
    f32 = jnp.float32
    o = jax.ShapeDtypeStruct
    return pl.pallas_call(
        _prep_body,
        grid=(_NRB,),
        in_specs=[
            pl.BlockSpec((_RB, _D), lambda i: (i, 0)),
            pl.BlockSpec((_D, _H), lambda i: (0, 0)),
            pl.BlockSpec((1, _H), lambda i: (0, 0)),
            pl.BlockSpec((2, _RB, 16), lambda i: (0, i, 0)),
        ],
        out_specs=[
            pl.BlockSpec((_RB, 32), lambda i: (i, 0)),
            pl.BlockSpec((_RB, 32), lambda i: (i, 0)),
            pl.BlockSpec((_RB, 32), lambda i: (i, 0)),
            pl.BlockSpec((_RB, 32), lambda i: (i, 0)),
            pl.BlockSpec((_RB, 32), lambda i: (i, 0)),
            pl.BlockSpec((_RB, _H), lambda i: (i, 0)),
        ],
        out_shape=[
            o((_NP, 32), f32), o((_NP, 32), f32),
            o((_NP, 32), f32), o((_NP, 32), f32),
            o((_NP, 32), f32), o((_NP, _H), f32),
        ],
    )


def _prop_body(src_h, dst2_h, u0_h, u1_h, hd0_h, hd1_h, d2_h,
               uf0_h, uf1_h,
               agg_sh, sidx0, sidx1, didx0, didx1, didx2, didx3,
               rows0, rows1, aggc, d2c, hdc, zc,
               semg0, semg1, semi0, semi1, sems):
    c = lax.axis_index("c")
    s = lax.axis_index("s")
    row0 = pl.multiple_of(s * _RPT, 8)
    ebase = s * _EPT
    sidxs = (sidx0, sidx1)
    didxs = (didx0, didx1, didx2, didx3)
    rowss = (rows0, rows1)
    semgs = (semg0, semg1)
    semis = (semi0, semi1)

    def _psl(p):
        return pl.ds(pl.multiple_of(row0 + p * _PB, 8), _PB)

    def _eoff(b):
        return pl.multiple_of(ebase + b * _BLK, 128)

    def _idx_issue(b):
        semi = semis[b % 2]
        pltpu.async_copy(src_h.at[pl.ds(_eoff(b), _BLK)], sidxs[b % 2], semi)
        pltpu.async_copy(
            dst2_h.at[pl.ds(pl.multiple_of(_eoff(b) // 128, 8), 8)],
            didxs[b % 4], semi)

    def _idx_wait(b):
        semi = semis[b % 2]
        pltpu.make_async_copy(
            src_h.at[pl.ds(0, _BLK)], sidxs[b % 2], semi).wait()
        pltpu.make_async_copy(dst2_h.at[pl.ds(0, 8)], didxs[b % 4], semi).wait()

    def _gather_issue(b):
        sidx, rows, semg = sidxs[b % 2], rowss[b % 2], semgs[b % 2]

        @pl.when(c == 0)
        def _():
            pltpu.async_copy(uf0_h.at[sidx], rows, semg)

        @pl.when(c == 1)
        def _():
            pltpu.async_copy(uf1_h.at[sidx], rows, semg)

    def _gather_wait(b):
        pltpu.make_async_copy(
            uf0_h.at[sidxs[b % 2]], rowss[b % 2], semgs[b % 2]).wait()

    def _scatter_issue(b):
        rows, didx = rowss[b % 2], didxs[b % 4]
        for j in range(8):
            pltpu.async_copy(rows.at[pl.ds(j * 128, 128)],
                             agg_sh.at[didx.at[j]], sems, add=True)

    def _scatter_drain(b):
        rows, didx = rowss[b % 2], didxs[b % 4]
        for j in range(8):
            pltpu.make_async_copy(rows.at[pl.ds(j * 128, 128)],
                                  agg_sh.at[didx.at[j]], sems).wait()

    @pl.loop(0, _PB)
    def _(r):
        z16 = jnp.zeros((16,), jnp.float32)
        zc[r, pl.ds(0, 16)] = z16
        zc[r, pl.ds(16, 16)] = z16

    for p in range(_RPT // _PB):
        psl = _psl(p)

        @pl.when(c == 0)
        def _():
            pltpu.sync_copy(u0_h.at[psl], aggc)
            pltpu.sync_copy(aggc, uf0_h.at[psl])

        @pl.when(c == 1)
        def _():
            pltpu.sync_copy(u1_h.at[psl], aggc)
            pltpu.sync_copy(aggc, uf1_h.at[psl])

        pltpu.sync_copy(zc, agg_sh.at[psl])

    plsc.subcore_barrier()

    @pl.loop(0, _K)
    def _(k):
        pltpu.sync_copy(src_h.at[pl.ds(_eoff(0), _BLK)], sidx0)
        pltpu.sync_copy(
            dst2_h.at[pl.ds(pl.multiple_of(_eoff(0) // 128, 8), 8)], didx0)
        _gather_issue(0)
        _idx_issue(1)
        for b in range(_NBLK):
            if b >= 1:
                _scatter_drain(b - 1)
            if b + 1 < _NBLK:
                _idx_wait(b + 1)
                _gather_issue(b + 1)
            _gather_wait(b)
            _scatter_issue(b)
            if b + 2 < _NBLK:
                _idx_issue(b + 2)
        _scatter_drain(_NBLK - 1)

        plsc.subcore_barrier()
        for p in range(_RPT // _PB):
            psl = _psl(p)
            pltpu.sync_copy(agg_sh.at[psl], aggc)
            pltpu.sync_copy(d2_h.at[psl], d2c)

            @pl.when(c == 0)
            def _():
                pltpu.sync_copy(hd0_h.at[psl], hdc)

            @pl.when(c == 1)
            def _():
                pltpu.sync_copy(hd1_h.at[psl], hdc)

            @pl.loop(0, _PB)
            def _(r):
                for cc in (0, 16):
                    sl = pl.ds(cc, 16)
                    aggc[r, sl] = d2c[r, sl] * aggc[r, sl] + hdc[r, sl]

            @pl.when(c == 0)
            def _():
                pltpu.sync_copy(aggc, uf0_h.at[psl])

            @pl.when(c == 1)
            def _():
                pltpu.sync_copy(aggc, uf1_h.at[psl])

            pltpu.sync_copy(zc, agg_sh.at[psl])

        plsc.subcore_barrier()


def _make_prop():
    f32 = jnp.float32
    o = jax.ShapeDtypeStruct
    return pl.kernel(
        _prop_body,
        out_type=[o((_NP, 32), f32), o((_NP, 32), f32)],
        mesh=_sc_mesh(),
        compiler_params=pltpu.CompilerParams(use_tc_tiling_on_sc=False),
        scratch_types=[
            pltpu.VMEM_SHARED((_NP, 32), f32),
            pltpu.VMEM((_BLK,), jnp.int32),
            pltpu.VMEM((_BLK,), jnp.int32),
            pltpu.VMEM((8, 128), jnp.int32),
            pltpu.VMEM((8, 128), jnp.int32),
            pltpu.VMEM((8, 128), jnp.int32),
            pltpu.VMEM((8, 128), jnp.int32),
            pltpu.VMEM((_BLK, 32), f32),
            pltpu.VMEM((_BLK, 32), f32),
            pltpu.VMEM((_PB, 32), f32),
            pltpu.VMEM((_PB, 32), f32),
            pltpu.VMEM((_PB, 32), f32),
            pltpu.VMEM((_PB, 32), f32),
            pltpu.SemaphoreType.DMA,
            pltpu.SemaphoreType.DMA,
            pltpu.SemaphoreType.DMA,
            pltpu.SemaphoreType.DMA,
            pltpu.SemaphoreType.DMA,
        ],
    )


def _final_body(u0b, u1b, sdb, w2, b2r, lsm, xo, sm):
    z = jnp.concatenate([u0b[...], u1b[...]], axis=1) * sdb[...]
    logits = jnp.dot(z, w2[...], preferred_element_type=jnp.float32) + b2r[...]
    m = jnp.max(logits, axis=1, keepdims=True)
    ex = jnp.exp(logits - m)
    ssum = jnp.sum(ex, axis=1, keepdims=True)
    xo[...] = logits
    lsm[...] = logits - m - jnp.log(ssum)
    sm[...] = ex / ssum


def _make_final():
    f32 = jnp.float32
    o = jax.ShapeDtypeStruct
    return pl.pallas_call(
        _final_body,
        grid=(_NRB,),
        in_specs=[
            pl.BlockSpec((_RB, 32), lambda i: (i, 0)),
            pl.BlockSpec((_RB, 32), lambda i: (i, 0)),
            pl.BlockSpec((_RB, _H), lambda i: (i, 0)),
            pl.BlockSpec((_H, 128), lambda i: (0, 0)),
            pl.BlockSpec((1, 128), lambda i: (0, 0)),
        ],
        out_specs=[
            pl.BlockSpec((_RB, 128), lambda i: (i, 0)),
            pl.BlockSpec((_RB, 128), lambda i: (i, 0)),
            pl.BlockSpec((_RB, 128), lambda i: (i, 0)),
        ],
        out_shape=[o((_NP, 128), f32), o((_NP, 128), f32), o((_NP, 128), f32)],
    )


def kernel(x, edge_index, e_w, idx, W1, b1, W2, b2):
    del e_w, idx
    n_extra = _EP - (edge_index.shape[1] + _N)
    loops = jnp.arange(_N, dtype=jnp.int32)
    padv = _N + (jnp.arange(n_extra, dtype=jnp.int32) % (_NP - _N))
    src = jnp.concatenate([edge_index[0], loops, padv])
    dst = jnp.concatenate([edge_index[1], loops, padv])
    dst2 = dst.reshape(_DROWS, 128)

    xp = jnp.pad(x, ((0, _NP - _N), (0, 0)))
    b1r = b1.reshape(1, _H)
    w2p = jnp.pad(W2, ((0, 0), (0, 128 - _CLS)))
    b2r = jnp.concatenate(
        [b2, jnp.full((128 - _CLS,), -1e30, jnp.float32)]).reshape(1, 128)

    degp = _make_deg()(dst2).reshape(2, _NP, 16)
    u0, u1, hd0, hd1, d2, sd = _make_prep()(xp, W1, b1r, degp)
    uf0, uf1 = _make_prop()(src, dst2, u0, u1, hd0, hd1, d2)
    lsm, xo, sm = _make_final()(uf0, uf1, sd, w2p, b2r)
    return (lsm[:_N, :_CLS], xo[:_N, :_CLS], 0.0, sm[:_N, :_CLS])
